# baseline (speedup 1.0000x reference)
import jax
import jax.numpy as jnp
from jax import lax
from jax.experimental import pallas as pl
from jax.experimental.pallas import tpu as pltpu
from jax.experimental.pallas import tpu_sc as plsc

NN = 13
NCAT = 26
D = 64
T = 50
B = 1024
CARD = 100000
TOK = NN + NCAT
OUT_ROWS = 1 + T * TOK
NPAIRS = T * TOK
PW = 61


def _body(tbl_hbm, xall_hbm, numw_hbm, numb_hbm, catb_hbm, pos_hbm, cls_hbm,
          out_hbm, xrow_v, idx_v, stage_v, slab_v, numw_v, numb_v, catb_v,
          pos_v, cls_v, sem_x, sem_g, sem_s):
    wid = lax.axis_index("s") * 2 + lax.axis_index("c")
    p0 = wid * PW
    npairs = jnp.where(wid == 31, NPAIRS - 31 * PW, PW)

    pltpu.sync_copy(numw_hbm, numw_v)
    pltpu.sync_copy(numb_hbm, numb_v)
    pltpu.sync_copy(catb_hbm, catb_v)
    pltpu.sync_copy(pos_hbm, pos_v)
    pltpu.sync_copy(cls_hbm, cls_v)

    ridx = [lax.iota(jnp.int32, 16) + 16 * bl for bl in range(8)]

    def wait_store():
        pltpu.make_async_copy(
            slab_v.at[0], out_hbm.at[0, :, pl.ds(0, 128)], sem_s).wait()

    def fire_store(spar, r, gb):
        pltpu.async_copy(
            slab_v.at[spar], out_hbm.at[r, :, pl.ds(gb * 128, 128)], sem_s)

    @pl.when(wid == 31)
    def _cls():
        cl = [cls_v[0, pl.ds(16 * k, 16)] for k in range(4)]
        for d in range(D):
            v16 = jnp.broadcast_to(cl[d // 16][d % 16], (16,))
            for par in range(2):
                for bl in range(8):
                    slab_v[par, d, pl.ds(bl * 16, 16)] = v16
        for gb in range(8):
            if gb >= 2:
                wait_store()
            fire_store(gb % 2, 0, gb)

    q0 = jnp.where(wid == 31, 8, 0).astype(jnp.int32)

    def _tf(i, c):
        t, f = c
        big = (f >= TOK).astype(jnp.int32)
        return (t + big, f - big * TOK)

    t0, f0 = lax.fori_loop(0, T, _tf, (jnp.int32(0), p0.astype(jnp.int32)))

    pltpu.async_copy(xall_hbm.at[1 + p0], xrow_v.at[0], sem_x)

    def num_fn(t, f, q, r, xpar):
        wv = [numw_v[f, pl.ds(16 * k, 16)] for k in range(4)]
        cv = [numb_v[f, pl.ds(16 * k, 16)] + pos_v[t, pl.ds(16 * k, 16)]
              for k in range(4)]

        def gb_body(gb, q):
            spar = q & 1

            @pl.when(q >= 2)
            def _():
                wait_store()

            xv = [xrow_v[xpar, gb, pl.ds(16 * bl, 16)] for bl in range(8)]
            for d in range(D):
                ws = wv[d // 16][d % 16]
                cs = cv[d // 16][d % 16]
                for bl in range(8):
                    slab_v[spar, d, pl.ds(16 * bl, 16)] = xv[bl] * ws + cs
            fire_store(spar, r, gb)
            return q + 1

        return lax.fori_loop(0, 8, gb_body, q)

    def cat_fn(t, f, q, r, xpar):
        fc = f - NN
        off = fc * CARD
        cv = [catb_v[fc, pl.ds(16 * k, 16)] + pos_v[t, pl.ds(16 * k, 16)]
              for k in range(4)]
        for gb in range(8):
            for bl in range(8):
                idx_v[gb, pl.ds(16 * bl, 16)] = (
                    xrow_v[xpar, gb, pl.ds(16 * bl, 16)].astype(jnp.int32)
                    + off)
        pltpu.async_copy(tbl_hbm.at[idx_v.at[0]], stage_v.at[0], sem_g)

        def gb_body(gb, q):
            gpar = gb & 1
            pltpu.make_async_copy(
                tbl_hbm.at[idx_v.at[gb]], stage_v.at[gpar], sem_g).wait()

            @pl.when(gb < 7)
            def _():
                pltpu.async_copy(
                    tbl_hbm.at[idx_v.at[gb + 1]], stage_v.at[1 - gpar], sem_g)

            spar = q & 1

            @pl.when(q >= 2)
            def _():
                wait_store()

            for d in range(D):
                cs = cv[d // 16][d % 16]
                cidx = jnp.full((16,), d, jnp.int32)
                for bl in range(8):
                    v = plsc.load_gather(stage_v.at[gpar], [ridx[bl], cidx])
                    slab_v[spar, d, pl.ds(16 * bl, 16)] = v + cs
            fire_store(spar, r, gb)
            return q + 1

        return lax.fori_loop(0, 8, gb_body, q)

    def pair_body(i, carry):
        t, f, q = carry
        r = 1 + p0 + i
        xpar = i & 1
        pltpu.make_async_copy(
            xall_hbm.at[r], xrow_v.at[xpar], sem_x).wait()

        @pl.when(i < npairs - 1)
        def _():
            pltpu.async_copy(xall_hbm.at[r + 1], xrow_v.at[1 - xpar], sem_x)

        q = lax.cond(f < NN, num_fn, cat_fn, t, f, q, r, xpar)
        f = f + 1
        roll = (f == TOK).astype(jnp.int32)
        return (t + roll, f - roll * TOK, q)

    lax.fori_loop(0, npairs, pair_body, (t0, f0, q0))

    wait_store()
    wait_store()


@jax.jit
def _tokenize(tblp, xall, numw, numb, catb, pos, cls2):
    mesh = plsc.VectorSubcoreMesh(core_axis_name="c", subcore_axis_name="s")
    f = pl.kernel(
        _body,
        mesh=mesh,
        compiler_params=pltpu.CompilerParams(needs_layout_passes=False),
        out_type=jax.ShapeDtypeStruct((OUT_ROWS, D, B), jnp.float32),
        scratch_types=[
            pltpu.VMEM((2, 8, 128), jnp.float32),
            pltpu.VMEM((8, 128), jnp.int32),
            pltpu.VMEM((2, 128, 128), jnp.float32),
            pltpu.VMEM((2, D, 128), jnp.float32),
            pltpu.VMEM((16, 128), jnp.float32),
            pltpu.VMEM((16, 128), jnp.float32),
            pltpu.VMEM((32, 128), jnp.float32),
            pltpu.VMEM((56, 128), jnp.float32),
            pltpu.VMEM((8, 128), jnp.float32),
            pltpu.SemaphoreType.DMA,
            pltpu.SemaphoreType.DMA,
            pltpu.SemaphoreType.DMA,
        ],
    )
    return f(tblp, xall, numw, numb, catb, pos, cls2)


def kernel(x_seq, num_weight, num_bias, cat_table, cat_bias, cls_token,
           pos_emb):
    tblp = jnp.pad(cat_table, ((0, 0), (0, 64)))
    xall = jnp.concatenate(
        [jnp.zeros((1, B), jnp.float32),
         x_seq.transpose(1, 2, 0).reshape(NPAIRS, B)], axis=0).reshape(
             OUT_ROWS, 8, 128)
    numw = jnp.pad(num_weight, ((0, 3), (0, 64)))
    numb = jnp.pad(num_bias, ((0, 3), (0, 64)))
    catb = jnp.pad(cat_bias, ((0, 6), (0, 64)))
    pos = jnp.pad(pos_emb, ((0, 6), (0, 64)))
    cls2 = jnp.pad(cls_token[None, :], ((0, 7), (0, 64)))
    out_phys = _tokenize(tblp, xall, numw, numb, catb, pos, cls2)
    return jnp.transpose(out_phys, (2, 0, 1))


# speedup vs baseline: 1.0024x; 1.0024x over previous
import jax
import jax.numpy as jnp
from jax import lax
from jax.experimental import pallas as pl
from jax.experimental.pallas import tpu as pltpu
from jax.experimental.pallas import tpu_sc as plsc

NN = 13
NCAT = 26
D = 64
T = 50
B = 1024
CARD = 100000
TOK = NN + NCAT
OUT_ROWS = 1 + T * TOK
NPAIRS = T * TOK
PW = 61


def _body(tbl_hbm, xall_hbm, numw_hbm, numb_hbm, catb_hbm, pos_hbm, cls_hbm,
          out_hbm, xrow_v, idx_v, stage_v, slab_v, numw_v, numb_v, catb_v,
          pos_v, cls_v, sem_x, sem_g, sem_s):
    wid = lax.axis_index("s") * 2 + lax.axis_index("c")
    p0 = wid * PW
    npairs = jnp.where(wid == 31, NPAIRS - 31 * PW, PW)

    pltpu.sync_copy(numw_hbm, numw_v)
    pltpu.sync_copy(numb_hbm, numb_v)
    pltpu.sync_copy(catb_hbm, catb_v)
    pltpu.sync_copy(pos_hbm, pos_v)
    pltpu.sync_copy(cls_hbm, cls_v)

    ridx = [lax.iota(jnp.int32, 16) + 16 * bl for bl in range(8)]

    def wait_store():
        pltpu.make_async_copy(
            slab_v.at[0], out_hbm.at[0, :, pl.ds(0, 128)], sem_s).wait()

    def fire_store(spar, r, gb):
        pltpu.async_copy(
            slab_v.at[spar], out_hbm.at[r, :, pl.ds(gb * 128, 128)], sem_s)

    @pl.when(wid == 31)
    def _cls():
        cl = [cls_v[0, pl.ds(16 * k, 16)] for k in range(4)]
        for d in range(D):
            v16 = jnp.broadcast_to(cl[d // 16][d % 16], (16,))
            for par in range(2):
                for bl in range(8):
                    slab_v[par, d, pl.ds(bl * 16, 16)] = v16
        for gb in range(8):
            if gb >= 2:
                wait_store()
            fire_store(gb % 2, 0, gb)

    q0 = jnp.where(wid == 31, 8, 0).astype(jnp.int32)

    def _tf(i, c):
        t, f = c
        big = (f >= TOK).astype(jnp.int32)
        return (t + big, f - big * TOK)

    t0, f0 = lax.fori_loop(0, T, _tf, (jnp.int32(0), p0.astype(jnp.int32)))

    pltpu.async_copy(xall_hbm.at[1 + p0], xrow_v.at[0], sem_x)

    def num_fn(t, f, q, r, xpar):
        wv = [numw_v[f, pl.ds(16 * k, 16)] for k in range(4)]
        cv = [numb_v[f, pl.ds(16 * k, 16)] + pos_v[t, pl.ds(16 * k, 16)]
              for k in range(4)]

        def gb_body(gb, q):
            spar = q & 1

            @pl.when(q >= 2)
            def _():
                wait_store()

            xv = [xrow_v[xpar, gb, pl.ds(16 * bl, 16)] for bl in range(8)]
            for d in range(D):
                ws = wv[d // 16][d % 16]
                cs = cv[d // 16][d % 16]
                for bl in range(8):
                    slab_v[spar, d, pl.ds(16 * bl, 16)] = xv[bl] * ws + cs
            fire_store(spar, r, gb)
            return q + 1

        return lax.fori_loop(0, 8, gb_body, q)

    def cat_fn(t, f, q, r, xpar):
        fc = f - NN
        off = fc * CARD
        cv = [catb_v[fc, pl.ds(16 * k, 16)] + pos_v[t, pl.ds(16 * k, 16)]
              for k in range(4)]
        for gb in range(8):
            for bl in range(8):
                idx_v[gb, pl.ds(16 * bl, 16)] = (
                    xrow_v[xpar, gb, pl.ds(16 * bl, 16)].astype(jnp.int32)
                    + off)
        for g in range(4):
            pltpu.async_copy(tbl_hbm.at[idx_v.at[g]], stage_v.at[g], sem_g)

        def gb_body(gb, q):
            gpar = gb & 3
            pltpu.make_async_copy(
                tbl_hbm.at[idx_v.at[gb]], stage_v.at[gpar], sem_g).wait()
            spar = q & 1

            @pl.when(q >= 2)
            def _():
                wait_store()

            for d in range(D):
                cs = cv[d // 16][d % 16]
                cidx = jnp.full((16,), d, jnp.int32)
                for bl in range(8):
                    v = plsc.load_gather(stage_v.at[gpar], [ridx[bl], cidx])
                    slab_v[spar, d, pl.ds(16 * bl, 16)] = v + cs
            fire_store(spar, r, gb)

            @pl.when(gb < 4)
            def _():
                pltpu.async_copy(
                    tbl_hbm.at[idx_v.at[gb + 4]], stage_v.at[gpar], sem_g)

            return q + 1

        return lax.fori_loop(0, 8, gb_body, q)

    def pair_body(i, carry):
        t, f, q = carry
        r = 1 + p0 + i
        xpar = i & 1
        pltpu.make_async_copy(
            xall_hbm.at[r], xrow_v.at[xpar], sem_x).wait()

        @pl.when(i < npairs - 1)
        def _():
            pltpu.async_copy(xall_hbm.at[r + 1], xrow_v.at[1 - xpar], sem_x)

        q = lax.cond(f < NN, num_fn, cat_fn, t, f, q, r, xpar)
        f = f + 1
        roll = (f == TOK).astype(jnp.int32)
        return (t + roll, f - roll * TOK, q)

    lax.fori_loop(0, npairs, pair_body, (t0, f0, q0))

    wait_store()
    wait_store()


@jax.jit
def _tokenize(tblp, xall, numw, numb, catb, pos, cls2):
    mesh = plsc.VectorSubcoreMesh(core_axis_name="c", subcore_axis_name="s")
    f = pl.kernel(
        _body,
        mesh=mesh,
        compiler_params=pltpu.CompilerParams(needs_layout_passes=False),
        out_type=jax.ShapeDtypeStruct((OUT_ROWS, D, B), jnp.float32),
        scratch_types=[
            pltpu.VMEM((2, 8, 128), jnp.float32),
            pltpu.VMEM((8, 128), jnp.int32),
            pltpu.VMEM((4, 128, 128), jnp.float32),
            pltpu.VMEM((2, D, 128), jnp.float32),
            pltpu.VMEM((16, 128), jnp.float32),
            pltpu.VMEM((16, 128), jnp.float32),
            pltpu.VMEM((32, 128), jnp.float32),
            pltpu.VMEM((56, 128), jnp.float32),
            pltpu.VMEM((8, 128), jnp.float32),
            pltpu.SemaphoreType.DMA,
            pltpu.SemaphoreType.DMA,
            pltpu.SemaphoreType.DMA,
        ],
    )
    return f(tblp, xall, numw, numb, catb, pos, cls2)


def kernel(x_seq, num_weight, num_bias, cat_table, cat_bias, cls_token,
           pos_emb):
    tblp = jnp.pad(cat_table, ((0, 0), (0, 64)))
    xall = jnp.concatenate(
        [jnp.zeros((1, B), jnp.float32),
         x_seq.transpose(1, 2, 0).reshape(NPAIRS, B)], axis=0).reshape(
             OUT_ROWS, 8, 128)
    numw = jnp.pad(num_weight, ((0, 3), (0, 64)))
    numb = jnp.pad(num_bias, ((0, 3), (0, 64)))
    catb = jnp.pad(cat_bias, ((0, 6), (0, 64)))
    pos = jnp.pad(pos_emb, ((0, 6), (0, 64)))
    cls2 = jnp.pad(cls_token[None, :], ((0, 7), (0, 64)))
    out_phys = _tokenize(tblp, xall, numw, numb, catb, pos, cls2)
    return jnp.transpose(out_phys, (2, 0, 1))


# speedup vs baseline: 1.0681x; 1.0656x over previous
import jax
import jax.numpy as jnp
from jax import lax
from jax.experimental import pallas as pl
from jax.experimental.pallas import tpu as pltpu
from jax.experimental.pallas import tpu_sc as plsc

NN = 13
NCAT = 26
D = 64
T = 50
B = 1024
CARD = 100000
TOK = NN + NCAT
OUT_ROWS = 1 + T * TOK
NUMP = T * NN
CATP = T * NCAT
PWN = 21
PWC = 41


def _init_tf(p0, per):
    def _tf(i, c):
        t, f = c
        big = (f >= per).astype(jnp.int32)
        return (t + big, f - big * per)

    return lax.fori_loop(0, T, _tf, (jnp.int32(0), p0.astype(jnp.int32)))


def _num_body(xall_hbm, numw_hbm, numb_hbm, pos_hbm, cls_hbm, out_hbm,
              xrow_v, slab_v, numw_v, numb_v, pos_v, cls_v, sem_x, sem_s):
    wid = lax.axis_index("s") * 2 + lax.axis_index("c")
    p0 = wid * PWN
    npairs = jnp.clip(NUMP - p0, 0, PWN)

    pltpu.sync_copy(numw_hbm, numw_v)
    pltpu.sync_copy(numb_hbm, numb_v)
    pltpu.sync_copy(pos_hbm, pos_v)
    pltpu.sync_copy(cls_hbm, cls_v)

    def wait_store():
        pltpu.make_async_copy(
            slab_v.at[0], out_hbm.at[0, :, pl.ds(0, 128)], sem_s).wait()

    def fire_store(spar, r, gb):
        pltpu.async_copy(
            slab_v.at[spar], out_hbm.at[r, :, pl.ds(gb * 128, 128)], sem_s)

    @pl.when(wid == 31)
    def _cls():
        cl = [cls_v[0, pl.ds(16 * k, 16)] for k in range(4)]
        for d in range(D):
            v16 = jnp.broadcast_to(cl[d // 16][d % 16], (16,))
            for par in range(2):
                for bl in range(8):
                    slab_v[par, d, pl.ds(bl * 16, 16)] = v16
        for gb in range(8):
            if gb >= 2:
                wait_store()
            fire_store(gb % 2, 0, gb)

    q0 = jnp.where(wid == 31, 8, 0).astype(jnp.int32)
    t0, f0 = _init_tf(p0, NN)

    @pl.when(npairs > 0)
    def _prefetch():
        pltpu.async_copy(
            xall_hbm.at[1 + t0 * TOK + f0], xrow_v.at[0], sem_x)

    def pair_body(i, carry):
        t, f, q = carry
        r = 1 + t * TOK + f
        xpar = i & 1
        pltpu.make_async_copy(
            xall_hbm.at[r], xrow_v.at[xpar], sem_x).wait()
        fn = f + 1
        roll = (fn == NN).astype(jnp.int32)
        tn, fn = t + roll, fn - roll * NN

        @pl.when(i < npairs - 1)
        def _():
            pltpu.async_copy(
                xall_hbm.at[1 + tn * TOK + fn], xrow_v.at[1 - xpar], sem_x)

        wv = [numw_v[f, pl.ds(16 * k, 16)] for k in range(4)]
        cv = [numb_v[f, pl.ds(16 * k, 16)] + pos_v[t, pl.ds(16 * k, 16)]
              for k in range(4)]

        def gb_body(gb, q):
            spar = q & 1

            @pl.when(q >= 2)
            def _():
                wait_store()

            xv = [xrow_v[xpar, gb, pl.ds(16 * bl, 16)] for bl in range(8)]
            for d in range(D):
                ws = wv[d // 16][d % 16]
                cs = cv[d // 16][d % 16]
                for bl in range(8):
                    slab_v[spar, d, pl.ds(16 * bl, 16)] = xv[bl] * ws + cs
            fire_store(spar, r, gb)
            return q + 1

        q = lax.fori_loop(0, 8, gb_body, q)
        return (tn, fn, q)

    _, _, qf = lax.fori_loop(0, npairs, pair_body, (t0, f0, q0))

    @pl.when(qf >= 1)
    def _d1():
        wait_store()

    @pl.when(qf >= 2)
    def _d2():
        wait_store()


def _cat_body(tbl_hbm, xall_hbm, catb_hbm, pos_hbm, out_hbm,
              xrow_v, idx_v, stage_v, slab_v, catb_v, pos_v, sem_x, sem_g,
              sem_s):
    wid = lax.axis_index("s") * 2 + lax.axis_index("c")
    p0 = wid * PWC
    npairs = jnp.clip(CATP - p0, 0, PWC)

    pltpu.sync_copy(catb_hbm, catb_v)
    pltpu.sync_copy(pos_hbm, pos_v)

    ridx = [lax.iota(jnp.int32, 16) + 16 * bl for bl in range(8)]

    def wait_store():
        pltpu.make_async_copy(
            slab_v.at[0], out_hbm.at[0, :, pl.ds(0, 128)], sem_s).wait()

    def fire_store(spar, r, gb):
        pltpu.async_copy(
            slab_v.at[spar], out_hbm.at[r, :, pl.ds(gb * 128, 128)], sem_s)

    t0, f0 = _init_tf(p0, NCAT)

    @pl.when(npairs > 0)
    def _prefetch():
        pltpu.async_copy(
            xall_hbm.at[1 + t0 * TOK + NN + f0], xrow_v.at[0], sem_x)

    def pair_body(i, carry):
        t, f, q = carry
        r = 1 + t * TOK + NN + f
        xpar = i & 1
        pltpu.make_async_copy(
            xall_hbm.at[r], xrow_v.at[xpar], sem_x).wait()
        fn = f + 1
        roll = (fn == NCAT).astype(jnp.int32)
        tn, fn = t + roll, fn - roll * NCAT

        @pl.when(i < npairs - 1)
        def _():
            pltpu.async_copy(
                xall_hbm.at[1 + tn * TOK + NN + fn], xrow_v.at[1 - xpar],
                sem_x)

        off = f * CARD
        cv = [catb_v[f, pl.ds(16 * k, 16)] + pos_v[t, pl.ds(16 * k, 16)]
              for k in range(4)]
        for gb in range(8):
            for bl in range(8):
                idx_v[gb, pl.ds(16 * bl, 16)] = (
                    xrow_v[xpar, gb, pl.ds(16 * bl, 16)].astype(jnp.int32)
                    + off)
        for g in range(4):
            pltpu.async_copy(tbl_hbm.at[idx_v.at[g]], stage_v.at[g], sem_g)

        def gb_body(gb, q):
            gpar = gb & 3
            pltpu.make_async_copy(
                tbl_hbm.at[idx_v.at[gb]], stage_v.at[gpar], sem_g).wait()
            spar = q & 1

            @pl.when(q >= 2)
            def _():
                wait_store()

            for d in range(D):
                cs = cv[d // 16][d % 16]
                cidx = jnp.full((16,), d, jnp.int32)
                for bl in range(8):
                    v = plsc.load_gather(stage_v.at[gpar], [ridx[bl], cidx])
                    slab_v[spar, d, pl.ds(16 * bl, 16)] = v + cs
            fire_store(spar, r, gb)

            @pl.when(gb < 4)
            def _():
                pltpu.async_copy(
                    tbl_hbm.at[idx_v.at[gb + 4]], stage_v.at[gpar], sem_g)

            return q + 1

        q = lax.fori_loop(0, 8, gb_body, q)
        return (tn, fn, q)

    _, _, qf = lax.fori_loop(0, npairs, pair_body, (t0, f0, jnp.int32(0)))

    @pl.when(qf >= 1)
    def _d1():
        wait_store()

    @pl.when(qf >= 2)
    def _d2():
        wait_store()


@jax.jit
def _tokenize(tblp, xall, numw, numb, catb, pos, cls2):
    mesh = plsc.VectorSubcoreMesh(core_axis_name="c", subcore_axis_name="s")
    out_ref = jax.new_ref(
        jnp.zeros((OUT_ROWS, D, B), jnp.float32))

    num_k = pl.kernel(
        _num_body,
        mesh=mesh,
        compiler_params=pltpu.CompilerParams(needs_layout_passes=False),
        out_type=(),
        scratch_types=[
            pltpu.VMEM((2, 8, 128), jnp.float32),
            pltpu.VMEM((2, D, 128), jnp.float32),
            pltpu.VMEM((16, 128), jnp.float32),
            pltpu.VMEM((16, 128), jnp.float32),
            pltpu.VMEM((56, 128), jnp.float32),
            pltpu.VMEM((8, 128), jnp.float32),
            pltpu.SemaphoreType.DMA,
            pltpu.SemaphoreType.DMA,
        ],
    )
    cat_k = pl.kernel(
        _cat_body,
        mesh=mesh,
        compiler_params=pltpu.CompilerParams(needs_layout_passes=False),
        out_type=(),
        scratch_types=[
            pltpu.VMEM((2, 8, 128), jnp.float32),
            pltpu.VMEM((8, 128), jnp.int32),
            pltpu.VMEM((4, 128, 128), jnp.float32),
            pltpu.VMEM((2, D, 128), jnp.float32),
            pltpu.VMEM((32, 128), jnp.float32),
            pltpu.VMEM((56, 128), jnp.float32),
            pltpu.SemaphoreType.DMA,
            pltpu.SemaphoreType.DMA,
            pltpu.SemaphoreType.DMA,
        ],
    )
    num_k(xall, numw, numb, pos, cls2, out_ref)
    cat_k(tblp, xall, catb, pos, out_ref)
    return out_ref[...]


def kernel(x_seq, num_weight, num_bias, cat_table, cat_bias, cls_token,
           pos_emb):
    tblp = jnp.pad(cat_table, ((0, 0), (0, 64)))
    xall = jnp.concatenate(
        [jnp.zeros((1, B), jnp.float32),
         x_seq.transpose(1, 2, 0).reshape(T * TOK, B)], axis=0).reshape(
             OUT_ROWS, 8, 128)
    numw = jnp.pad(num_weight, ((0, 3), (0, 64)))
    numb = jnp.pad(num_bias, ((0, 3), (0, 64)))
    catb = jnp.pad(cat_bias, ((0, 6), (0, 64)))
    pos = jnp.pad(pos_emb, ((0, 6), (0, 64)))
    cls2 = jnp.pad(cls_token[None, :], ((0, 7), (0, 64)))
    out_phys = _tokenize(tblp, xall, numw, numb, catb, pos, cls2)
    return jnp.transpose(out_phys, (2, 0, 1))
